# Initial kernel scaffold; baseline (speedup 1.0000x reference)
import jax
import jax.numpy as jnp
from jax.experimental import pallas as pl


def kernel(x, adj, W1, W2, W3, W4, W5, W6):
    raise NotImplementedError("write your pallas kernel here")


# speedup vs baseline: 1.0958x; 1.0958x over previous
import jax
import jax.numpy as jnp
from jax.experimental import pallas as pl

N = 4096
_BM = 256
_RM = 512
_RN = 2048


def _make_chain_kernel(n_w, out_f32):
    def body(a_ref, t_ref, *refs):
        ws = refs[:n_w]
        out_ref = refs[n_w]
        acc = jnp.dot(a_ref[...], t_ref[...], preferred_element_type=jnp.float32)
        for w in ws:
            acc = jnp.dot(acc.astype(jnp.bfloat16), w[...],
                          preferred_element_type=jnp.float32)
        out_ref[...] = acc if out_f32 else acc.astype(jnp.bfloat16)

    return body


def _mm_chain(a, t, ws, out_f32, bm=_BM):
    m, k = a.shape
    f_out = ws[-1].shape[1] if ws else t.shape[1]
    in_specs = [
        pl.BlockSpec((bm, k), lambda i: (i, 0)),
        pl.BlockSpec(t.shape, lambda i: (0, 0)),
    ]
    for w in ws:
        in_specs.append(pl.BlockSpec(w.shape, lambda i: (0, 0)))
    out_dtype = jnp.float32 if out_f32 else jnp.bfloat16
    return pl.pallas_call(
        _make_chain_kernel(len(ws), out_f32),
        grid=(m // bm,),
        in_specs=in_specs,
        out_specs=pl.BlockSpec((bm, f_out), lambda i: (i, 0)),
        out_shape=jax.ShapeDtypeStruct((m, f_out), out_dtype),
    )(a, t, *ws)


def _recon_kernel(zs_i, zs_j, zh_i, zh_j, o_ref):
    dn = (((1,), (1,)), ((), ()))
    a = jax.lax.dot_general(zs_i[...], zs_j[...], dn,
                            preferred_element_type=jnp.float32)
    b = jax.lax.dot_general(zh_i[...], zh_j[...], dn,
                            preferred_element_type=jnp.float32)
    o_ref[...] = jax.nn.sigmoid(a) + jax.nn.sigmoid(b)


def _recon(zs, zh):
    return pl.pallas_call(
        _recon_kernel,
        grid=(N // _RM, N // _RN),
        in_specs=[
            pl.BlockSpec((_RM, zs.shape[1]), lambda i, j: (i, 0)),
            pl.BlockSpec((_RN, zs.shape[1]), lambda i, j: (j, 0)),
            pl.BlockSpec((_RM, zh.shape[1]), lambda i, j: (i, 0)),
            pl.BlockSpec((_RN, zh.shape[1]), lambda i, j: (j, 0)),
        ],
        out_specs=pl.BlockSpec((_RM, _RN), lambda i, j: (i, j)),
        out_shape=jax.ShapeDtypeStruct((N, N), jnp.float32),
    )(zs, zs, zh, zh)


def kernel(x, adj, W1, W2, W3, W4, W5, W6):
    bf = jnp.bfloat16
    adjb = adj.astype(bf)
    xb = x.astype(bf)
    w1, w2, w3 = W1.astype(bf), W2.astype(bf), W3.astype(bf)
    w4, w5, w6 = W4.astype(bf), W5.astype(bf), W6.astype(bf)

    t1 = _mm_chain(xb, w1, [], out_f32=False)
    t2 = _mm_chain(adjb, t1, [w2], out_f32=False)
    t3 = _mm_chain(adjb, t2, [w3], out_f32=False)
    z_sgae = _mm_chain(adjb, t3, [], out_f32=True)
    zsb = z_sgae.astype(bf)

    z4 = _mm_chain(adjb, zsb, [w4], out_f32=False)
    t6 = _mm_chain(adjb, z4, [w5, w6], out_f32=False)
    z_hat = _mm_chain(adjb, t6, [], out_f32=True)
    zhb = z_hat.astype(bf)

    adj_hat = _recon(zsb, zhb)
    return (z_sgae, z_hat, adj_hat)


# speedup vs baseline: 1.5394x; 1.4048x over previous
import jax
import jax.numpy as jnp
from jax.experimental import pallas as pl

N = 4096
_BM = 512
_RM = 512


def _sigmoid(v):
    return 0.5 * jnp.tanh(0.5 * v) + 0.5


def _t1_kernel(x_ref, w_ref, o_ref):
    xb = x_ref[...].astype(jnp.bfloat16)
    o_ref[...] = jnp.dot(xb, w_ref[...],
                         preferred_element_type=jnp.float32).astype(jnp.bfloat16)


def _t2_kernel(adj_ref, t_ref, w_ref, o_ref, adjb_ref):
    ab = adj_ref[...].astype(jnp.bfloat16)
    adjb_ref[...] = ab
    acc = jnp.dot(ab, t_ref[...], preferred_element_type=jnp.float32)
    o_ref[...] = jnp.dot(acc.astype(jnp.bfloat16), w_ref[...],
                         preferred_element_type=jnp.float32).astype(jnp.bfloat16)


def _make_chain_kernel(n_w, emit_f32):
    def body(a_ref, t_ref, *refs):
        ws = refs[:n_w]
        outs = refs[n_w:]
        acc = jnp.dot(a_ref[...], t_ref[...], preferred_element_type=jnp.float32)
        for w in ws:
            acc = jnp.dot(acc.astype(jnp.bfloat16), w[...],
                          preferred_element_type=jnp.float32)
        if emit_f32:
            outs[0][...] = acc
            outs[1][...] = acc.astype(jnp.bfloat16)
        else:
            outs[0][...] = acc.astype(jnp.bfloat16)

    return body


def _mm_chain(a, t, ws, emit_f32, bm=_BM):
    m, k = a.shape
    f_out = ws[-1].shape[1] if ws else t.shape[1]
    in_specs = [
        pl.BlockSpec((bm, k), lambda i: (i, 0)),
        pl.BlockSpec(t.shape, lambda i: (0, 0)),
    ]
    for w in ws:
        in_specs.append(pl.BlockSpec(w.shape, lambda i: (0, 0)))
    if emit_f32:
        out_shape = (jax.ShapeDtypeStruct((m, f_out), jnp.float32),
                     jax.ShapeDtypeStruct((m, f_out), jnp.bfloat16))
        out_specs = (pl.BlockSpec((bm, f_out), lambda i: (i, 0)),
                     pl.BlockSpec((bm, f_out), lambda i: (i, 0)))
    else:
        out_shape = jax.ShapeDtypeStruct((m, f_out), jnp.bfloat16)
        out_specs = pl.BlockSpec((bm, f_out), lambda i: (i, 0))
    return pl.pallas_call(
        _make_chain_kernel(len(ws), emit_f32),
        grid=(m // bm,),
        in_specs=in_specs,
        out_specs=out_specs,
        out_shape=out_shape,
    )(a, t, *ws)


def _recon_kernel(zs_i, zs_all, zh_i, zh_all, o_ref):
    dn = (((1,), (1,)), ((), ()))
    a = jax.lax.dot_general(zs_i[...], zs_all[...], dn,
                            preferred_element_type=jnp.float32)
    b = jax.lax.dot_general(zh_i[...], zh_all[...], dn,
                            preferred_element_type=jnp.float32)
    o_ref[...] = _sigmoid(a) + _sigmoid(b)


def _recon(zs, zh):
    return pl.pallas_call(
        _recon_kernel,
        grid=(N // _RM,),
        in_specs=[
            pl.BlockSpec((_RM, zs.shape[1]), lambda i: (i, 0)),
            pl.BlockSpec(zs.shape, lambda i: (0, 0)),
            pl.BlockSpec((_RM, zh.shape[1]), lambda i: (i, 0)),
            pl.BlockSpec(zh.shape, lambda i: (0, 0)),
        ],
        out_specs=pl.BlockSpec((_RM, N), lambda i: (i, 0)),
        out_shape=jax.ShapeDtypeStruct((N, N), jnp.float32),
    )(zs, zs, zh, zh)


def kernel(x, adj, W1, W2, W3, W4, W5, W6):
    bf = jnp.bfloat16
    w1, w2, w3 = W1.astype(bf), W2.astype(bf), W3.astype(bf)
    w4, w5, w6 = W4.astype(bf), W5.astype(bf), W6.astype(bf)

    t1 = pl.pallas_call(
        _t1_kernel,
        grid=(N // _BM,),
        in_specs=[pl.BlockSpec((_BM, x.shape[1]), lambda i: (i, 0)),
                  pl.BlockSpec(w1.shape, lambda i: (0, 0))],
        out_specs=pl.BlockSpec((_BM, w1.shape[1]), lambda i: (i, 0)),
        out_shape=jax.ShapeDtypeStruct((N, w1.shape[1]), bf),
    )(x, w1)

    t2, adjb = pl.pallas_call(
        _t2_kernel,
        grid=(N // _BM,),
        in_specs=[pl.BlockSpec((_BM, N), lambda i: (i, 0)),
                  pl.BlockSpec(t1.shape, lambda i: (0, 0)),
                  pl.BlockSpec(w2.shape, lambda i: (0, 0))],
        out_specs=(pl.BlockSpec((_BM, w2.shape[1]), lambda i: (i, 0)),
                   pl.BlockSpec((_BM, N), lambda i: (i, 0))),
        out_shape=(jax.ShapeDtypeStruct((N, w2.shape[1]), bf),
                   jax.ShapeDtypeStruct((N, N), bf)),
    )(adj, t1, w2)

    t3 = _mm_chain(adjb, t2, [w3], emit_f32=False)
    z_sgae, zsb = _mm_chain(adjb, t3, [], emit_f32=True)

    z4 = _mm_chain(adjb, zsb, [w4], emit_f32=False)
    t6 = _mm_chain(adjb, z4, [w5, w6], emit_f32=False)
    z_hat, zhb = _mm_chain(adjb, t6, [], emit_f32=True)

    adj_hat = _recon(zsb, zhb)
    return (z_sgae, z_hat, adj_hat)


# speedup vs baseline: 1.5443x; 1.0032x over previous
import jax
import jax.numpy as jnp
from jax.experimental import pallas as pl
from jax.experimental.pallas import tpu as pltpu

N = 4096
_BM = 256
_NB = N // _BM
_RM = 512
_F32 = jnp.float32
_BF16 = jnp.bfloat16


def _mega_kernel(x_ref, adj_ref, w1, w2, w3, w4, w5, w6,
                 zs_out, zh_out, adjb, ta, tb, zsb):
    l = pl.program_id(0)
    i = pl.program_id(1)
    rows = pl.ds(i * _BM, _BM)

    def dot(a, b):
        return jnp.dot(a, b, preferred_element_type=_F32)

    @pl.when(l == 0)
    def _():
        xb = x_ref[...].astype(_BF16)
        ta[rows, :] = dot(xb, w1[...]).astype(_BF16)

    @pl.when(l == 1)
    def _():
        ab = adj_ref[...].astype(_BF16)
        adjb[rows, :] = ab
        acc = dot(ab, ta[...])
        tb[rows, :256] = dot(acc.astype(_BF16), w2[...]).astype(_BF16)

    @pl.when(l == 2)
    def _():
        acc = dot(adjb[rows, :], tb[:, :256])
        ta[rows, :128] = dot(acc.astype(_BF16), w3[...]).astype(_BF16)

    @pl.when(l == 3)
    def _():
        acc = dot(adjb[rows, :], ta[:, :128])
        zs_out[...] = acc
        zsb[rows, :] = acc.astype(_BF16)

    @pl.when(l == 4)
    def _():
        acc = dot(adjb[rows, :], zsb[...])
        tb[rows, :256] = dot(acc.astype(_BF16), w4[...]).astype(_BF16)

    @pl.when(l == 5)
    def _():
        acc = dot(adjb[rows, :], tb[:, :256])
        acc = dot(acc.astype(_BF16), w5[...])
        ta[rows, :512] = dot(acc.astype(_BF16), w6[...]).astype(_BF16)

    @pl.when(l == 6)
    def _():
        zh_out[...] = dot(adjb[rows, :], ta[:, :512])


def _sigmoid(v):
    return 0.5 * jnp.tanh(0.5 * v) + 0.5


def _recon_kernel(zs_i, zs_all, zh_i, zh_all, o_ref):
    dn = (((1,), (1,)), ((), ()))
    zsr = zs_all[...].astype(_BF16)
    zhr = zh_all[...].astype(_BF16)
    a = jax.lax.dot_general(zs_i[...].astype(_BF16), zsr, dn,
                            preferred_element_type=_F32)
    b = jax.lax.dot_general(zh_i[...].astype(_BF16), zhr, dn,
                            preferred_element_type=_F32)
    o_ref[...] = _sigmoid(a) + _sigmoid(b)


def _recon(zs, zh):
    return pl.pallas_call(
        _recon_kernel,
        grid=(N // _RM,),
        in_specs=[
            pl.BlockSpec((_RM, zs.shape[1]), lambda i: (i, 0)),
            pl.BlockSpec(zs.shape, lambda i: (0, 0)),
            pl.BlockSpec((_RM, zh.shape[1]), lambda i: (i, 0)),
            pl.BlockSpec(zh.shape, lambda i: (0, 0)),
        ],
        out_specs=pl.BlockSpec((_RM, N), lambda i: (i, 0)),
        out_shape=jax.ShapeDtypeStruct((N, N), _F32),
    )(zs, zs, zh, zh)


def kernel(x, adj, W1, W2, W3, W4, W5, W6):
    ws = [w.astype(_BF16) for w in (W1, W2, W3, W4, W5, W6)]
    last = _NB - 1

    def _x_map(l, i):
        return (jnp.where(l == 0, i, last), 0)

    def _adj_map(l, i):
        return (jnp.where(l == 1, i, jnp.where(l < 1, 0, last)), 0)

    def _w_map(l, i):
        return (0, 0)

    def _zs_map(l, i):
        return (jnp.where(l == 3, i, jnp.where(l < 3, 0, last)), 0)

    def _zh_map(l, i):
        return (jnp.where(l == 6, i, 0), 0)

    in_specs = [
        pl.BlockSpec((_BM, x.shape[1]), _x_map),
        pl.BlockSpec((_BM, N), _adj_map),
    ]
    for w in ws:
        in_specs.append(pl.BlockSpec(w.shape, _w_map))

    z_sgae, z_hat = pl.pallas_call(
        _mega_kernel,
        grid=(7, _NB),
        in_specs=in_specs,
        out_specs=(pl.BlockSpec((_BM, 128), _zs_map),
                   pl.BlockSpec((_BM, 512), _zh_map)),
        out_shape=(jax.ShapeDtypeStruct((N, 128), _F32),
                   jax.ShapeDtypeStruct((N, 512), _F32)),
        scratch_shapes=[
            pltpu.VMEM((N, N), _BF16),
            pltpu.VMEM((N, 512), _BF16),
            pltpu.VMEM((N, 512), _BF16),
            pltpu.VMEM((N, 128), _BF16),
        ],
    )(x, adj, *ws)

    adj_hat = _recon(z_sgae, z_hat)
    return (z_sgae, z_hat, adj_hat)


# speedup vs baseline: 1.9364x; 1.2539x over previous
import jax
import jax.numpy as jnp
from jax.experimental import pallas as pl
from jax.experimental.pallas import tpu as pltpu

N = 4096
_BM = 256
_NB = N // _BM
_RM = 512
_F32 = jnp.float32
_BF16 = jnp.bfloat16


def _mega_kernel(x_ref, adj_ref, w1, w2, w3, w4, w5, w6,
                 zs_out, zh_out, adjb, ta, tb, zsb, pe, pd):
    l = pl.program_id(0)
    i = pl.program_id(1)
    rows = pl.ds(i * _BM, _BM)

    def dot(a, b):
        return jnp.dot(a, b, preferred_element_type=_F32)

    @pl.when(l == 0)
    def _():
        @pl.when(i == 0)
        def _():
            e = dot(w1[...].astype(_BF16), w2[...].astype(_BF16))
            e = dot(e.astype(_BF16), w3[...].astype(_BF16))
            pe[...] = e.astype(_BF16)
            d = dot(w4[...].astype(_BF16), w5[...].astype(_BF16))
            d = dot(d.astype(_BF16), w6[...].astype(_BF16))
            pd[...] = d.astype(_BF16)

        xb = x_ref[...].astype(_BF16)
        ta[rows, :] = dot(xb, pe[...]).astype(_BF16)

    @pl.when(l == 1)
    def _():
        ab = adj_ref[...].astype(_BF16)
        adjb[rows, :] = ab
        tb[rows, :] = dot(ab, ta[...]).astype(_BF16)

    @pl.when(l == 2)
    def _():
        ta[rows, :] = dot(adjb[rows, :], tb[...]).astype(_BF16)

    @pl.when(l == 3)
    def _():
        acc = dot(adjb[rows, :], ta[...])
        zs_out[...] = acc
        zsb[rows, :] = acc.astype(_BF16)

    @pl.when(l == 4)
    def _():
        tb[rows, :] = dot(adjb[rows, :], zsb[...]).astype(_BF16)

    @pl.when(l == 5)
    def _():
        ta[rows, :] = dot(adjb[rows, :], tb[...]).astype(_BF16)

    @pl.when(l == 6)
    def _():
        acc = dot(adjb[rows, :], ta[...])
        zh_out[...] = dot(acc.astype(_BF16), pd[...])


def _sigmoid(v):
    return 0.5 * jnp.tanh(0.5 * v) + 0.5


def _recon_kernel(zs_i, zs_all, zh_i, zh_all, o_ref):
    dn = (((1,), (1,)), ((), ()))
    zsr = zs_all[...].astype(_BF16)
    zhr = zh_all[...].astype(_BF16)
    a = jax.lax.dot_general(zs_i[...].astype(_BF16), zsr, dn,
                            preferred_element_type=_F32)
    b = jax.lax.dot_general(zh_i[...].astype(_BF16), zhr, dn,
                            preferred_element_type=_F32)
    o_ref[...] = _sigmoid(a) + _sigmoid(b)


def _recon(zs, zh):
    return pl.pallas_call(
        _recon_kernel,
        grid=(N // _RM,),
        in_specs=[
            pl.BlockSpec((_RM, zs.shape[1]), lambda i: (i, 0)),
            pl.BlockSpec(zs.shape, lambda i: (0, 0)),
            pl.BlockSpec((_RM, zh.shape[1]), lambda i: (i, 0)),
            pl.BlockSpec(zh.shape, lambda i: (0, 0)),
        ],
        out_specs=pl.BlockSpec((_RM, N), lambda i: (i, 0)),
        out_shape=jax.ShapeDtypeStruct((N, N), _F32),
    )(zs, zs, zh, zh)


def kernel(x, adj, W1, W2, W3, W4, W5, W6):
    last = _NB - 1

    def _x_map(l, i):
        return (jnp.where(l == 0, i, last), 0)

    def _adj_map(l, i):
        return (jnp.where(l == 1, i, jnp.where(l < 1, 0, last)), 0)

    def _w_map(l, i):
        return (0, 0)

    def _zs_map(l, i):
        return (jnp.where(l == 3, i, jnp.where(l < 3, 0, last)), 0)

    def _zh_map(l, i):
        return (jnp.where(l == 6, i, 0), 0)

    ws = (W1, W2, W3, W4, W5, W6)
    in_specs = [
        pl.BlockSpec((_BM, x.shape[1]), _x_map),
        pl.BlockSpec((_BM, N), _adj_map),
    ]
    for w in ws:
        in_specs.append(pl.BlockSpec(w.shape, _w_map))

    z_sgae, z_hat = pl.pallas_call(
        _mega_kernel,
        grid=(7, _NB),
        in_specs=in_specs,
        out_specs=(pl.BlockSpec((_BM, 128), _zs_map),
                   pl.BlockSpec((_BM, 512), _zh_map)),
        out_shape=(jax.ShapeDtypeStruct((N, 128), _F32),
                   jax.ShapeDtypeStruct((N, 512), _F32)),
        scratch_shapes=[
            pltpu.VMEM((N, N), _BF16),
            pltpu.VMEM((N, 128), _BF16),
            pltpu.VMEM((N, 128), _BF16),
            pltpu.VMEM((N, 128), _BF16),
            pltpu.VMEM((512, 128), _BF16),
            pltpu.VMEM((128, 512), _BF16),
        ],
    )(x, adj, *ws)

    adj_hat = _recon(z_sgae, z_hat)
    return (z_sgae, z_hat, adj_hat)


# speedup vs baseline: 1.9843x; 1.0247x over previous
import jax
import jax.numpy as jnp
from jax.experimental import pallas as pl
from jax.experimental.pallas import tpu as pltpu

N = 4096
_BM = 256
_NB = N // _BM
_RM = 512
_F32 = jnp.float32
_BF16 = jnp.bfloat16


def _mega_kernel(x_ref, adj_ref, w1, w2, w3, w4, w5, w6,
                 zs_out, zh_out, zsb_out, zhb_out, adjb, ta, tb, zsb, pe, pd):
    l = pl.program_id(0)
    i = pl.program_id(1)
    rows = pl.ds(i * _BM, _BM)

    def dot(a, b):
        return jnp.dot(a, b, preferred_element_type=_F32)

    @pl.when(l == 0)
    def _():
        @pl.when(i == 0)
        def _():
            e = dot(w1[...].astype(_BF16), w2[...].astype(_BF16))
            e = dot(e.astype(_BF16), w3[...].astype(_BF16))
            pe[...] = e.astype(_BF16)
            d = dot(w4[...].astype(_BF16), w5[...].astype(_BF16))
            d = dot(d.astype(_BF16), w6[...].astype(_BF16))
            pd[...] = d.astype(_BF16)

        xb = x_ref[...].astype(_BF16)
        ta[rows, :] = dot(xb, pe[...]).astype(_BF16)

    @pl.when(l == 1)
    def _():
        ab = adj_ref[...].astype(_BF16)
        adjb[rows, :] = ab
        tb[rows, :] = dot(ab, ta[...]).astype(_BF16)

    @pl.when(l == 2)
    def _():
        ta[rows, :] = dot(adjb[rows, :], tb[...]).astype(_BF16)

    @pl.when(l == 3)
    def _():
        acc = dot(adjb[rows, :], ta[...])
        zs_out[...] = acc
        zb = acc.astype(_BF16)
        zsb_out[...] = zb
        zsb[rows, :] = zb

    @pl.when(l == 4)
    def _():
        tb[rows, :] = dot(adjb[rows, :], zsb[...]).astype(_BF16)

    @pl.when(l == 5)
    def _():
        ta[rows, :] = dot(adjb[rows, :], tb[...]).astype(_BF16)

    @pl.when(l == 6)
    def _():
        acc = dot(adjb[rows, :], ta[...])
        zh = dot(acc.astype(_BF16), pd[...])
        zh_out[...] = zh
        zhb_out[...] = zh.astype(_BF16)


def _sigmoid(v):
    return 0.5 * jnp.tanh(0.5 * v) + 0.5


def _recon_kernel(zs_i, zs_all, zh_i, zh_all, o_ref):
    dn = (((1,), (1,)), ((), ()))
    a = jax.lax.dot_general(zs_i[...], zs_all[...], dn,
                            preferred_element_type=_F32)
    b = jax.lax.dot_general(zh_i[...], zh_all[...], dn,
                            preferred_element_type=_F32)
    o_ref[...] = _sigmoid(a) + _sigmoid(b)


def _recon(zs, zh):
    return pl.pallas_call(
        _recon_kernel,
        grid=(N // _RM,),
        in_specs=[
            pl.BlockSpec((_RM, zs.shape[1]), lambda i: (i, 0)),
            pl.BlockSpec(zs.shape, lambda i: (0, 0)),
            pl.BlockSpec((_RM, zh.shape[1]), lambda i: (i, 0)),
            pl.BlockSpec(zh.shape, lambda i: (0, 0)),
        ],
        out_specs=pl.BlockSpec((_RM, N), lambda i: (i, 0)),
        out_shape=jax.ShapeDtypeStruct((N, N), _F32),
    )(zs, zs, zh, zh)


def kernel(x, adj, W1, W2, W3, W4, W5, W6):
    last = _NB - 1

    def _x_map(l, i):
        return (jnp.where(l == 0, i, last), 0)

    def _adj_map(l, i):
        return (jnp.where(l == 1, i, jnp.where(l < 1, 0, last)), 0)

    def _w_map(l, i):
        return (0, 0)

    def _zs_map(l, i):
        return (jnp.where(l == 3, i, jnp.where(l < 3, 0, last)), 0)

    def _zh_map(l, i):
        return (jnp.where(l == 6, i, 0), 0)

    ws = (W1, W2, W3, W4, W5, W6)
    in_specs = [
        pl.BlockSpec((_BM, x.shape[1]), _x_map),
        pl.BlockSpec((_BM, N), _adj_map),
    ]
    for w in ws:
        in_specs.append(pl.BlockSpec(w.shape, _w_map))

    z_sgae, z_hat, zsb16, zhb16 = pl.pallas_call(
        _mega_kernel,
        grid=(7, _NB),
        in_specs=in_specs,
        out_specs=(pl.BlockSpec((_BM, 128), _zs_map),
                   pl.BlockSpec((_BM, 512), _zh_map),
                   pl.BlockSpec((_BM, 128), _zs_map),
                   pl.BlockSpec((_BM, 512), _zh_map)),
        out_shape=(jax.ShapeDtypeStruct((N, 128), _F32),
                   jax.ShapeDtypeStruct((N, 512), _F32),
                   jax.ShapeDtypeStruct((N, 128), _BF16),
                   jax.ShapeDtypeStruct((N, 512), _BF16)),
        scratch_shapes=[
            pltpu.VMEM((N, N), _BF16),
            pltpu.VMEM((N, 128), _BF16),
            pltpu.VMEM((N, 128), _BF16),
            pltpu.VMEM((N, 128), _BF16),
            pltpu.VMEM((512, 128), _BF16),
            pltpu.VMEM((128, 512), _BF16),
        ],
    )(x, adj, *ws)

    adj_hat = _recon(zsb16, zhb16)
    return (z_sgae, z_hat, adj_hat)


# speedup vs baseline: 2.0659x; 1.0411x over previous
import jax
import jax.numpy as jnp
from jax.experimental import pallas as pl
from jax.experimental.pallas import tpu as pltpu

N = 4096
_BM = 256
_NB = N // _BM
_RM = 512
_F32 = jnp.float32
_BF16 = jnp.bfloat16


def _mega_kernel(x_ref, adj_ref, w1, w2, w3, w4, w5, w6,
                 zs_out, zh_out, zsb_out, mb_out, mg_out,
                 adjb, ta, tb, zsb, pe, pd, pg):
    l = pl.program_id(0)
    i = pl.program_id(1)
    rows = pl.ds(i * _BM, _BM)

    def dot(a, b):
        return jnp.dot(a, b, preferred_element_type=_F32)

    @pl.when(l == 0)
    def _():
        @pl.when(i == 0)
        def _():
            e = dot(dot(w1[...], w2[...]), w3[...])
            pe[...] = e.astype(_BF16)
            d = dot(dot(w4[...], w5[...]), w6[...])
            pd[...] = d.astype(_BF16)
            g = jax.lax.dot_general(d, d, (((1,), (1,)), ((), ())),
                                    preferred_element_type=_F32)
            pg[...] = g.astype(_BF16)

        xb = x_ref[...].astype(_BF16)
        ta[rows, :] = dot(xb, pe[...]).astype(_BF16)

    @pl.when(l == 1)
    def _():
        ab = adj_ref[...].astype(_BF16)
        adjb[rows, :] = ab
        tb[rows, :] = dot(ab, ta[...]).astype(_BF16)

    @pl.when(l == 2)
    def _():
        ta[rows, :] = dot(adjb[rows, :], tb[...]).astype(_BF16)

    @pl.when(l == 3)
    def _():
        acc = dot(adjb[rows, :], ta[...])
        zs_out[...] = acc
        zb = acc.astype(_BF16)
        zsb_out[...] = zb
        zsb[rows, :] = zb

    @pl.when(l == 4)
    def _():
        tb[rows, :] = dot(adjb[rows, :], zsb[...]).astype(_BF16)

    @pl.when(l == 5)
    def _():
        ta[rows, :] = dot(adjb[rows, :], tb[...]).astype(_BF16)

    @pl.when(l == 6)
    def _():
        m = dot(adjb[rows, :], ta[...])
        mb = m.astype(_BF16)
        zh_out[...] = dot(mb, pd[...])
        mb_out[...] = mb
        mg_out[...] = dot(mb, pg[...]).astype(_BF16)


def _sigmoid(v):
    return 0.5 * jnp.tanh(0.5 * v) + 0.5


def _recon_kernel(zs_i, zs_all, mg_i, m_all, o_ref):
    dn = (((1,), (1,)), ((), ()))
    a = jax.lax.dot_general(zs_i[...], zs_all[...], dn,
                            preferred_element_type=_F32)
    b = jax.lax.dot_general(mg_i[...], m_all[...], dn,
                            preferred_element_type=_F32)
    o_ref[...] = _sigmoid(a) + _sigmoid(b)


def _recon(zs, mg, m):
    return pl.pallas_call(
        _recon_kernel,
        grid=(N // _RM,),
        in_specs=[
            pl.BlockSpec((_RM, zs.shape[1]), lambda i: (i, 0)),
            pl.BlockSpec(zs.shape, lambda i: (0, 0)),
            pl.BlockSpec((_RM, mg.shape[1]), lambda i: (i, 0)),
            pl.BlockSpec(m.shape, lambda i: (0, 0)),
        ],
        out_specs=pl.BlockSpec((_RM, N), lambda i: (i, 0)),
        out_shape=jax.ShapeDtypeStruct((N, N), _F32),
    )(zs, zs, mg, m)


def kernel(x, adj, W1, W2, W3, W4, W5, W6):
    last = _NB - 1

    def _x_map(l, i):
        return (jnp.where(l == 0, i, last), 0)

    def _adj_map(l, i):
        return (jnp.where(l == 1, i, jnp.where(l < 1, 0, last)), 0)

    def _w_map(l, i):
        return (0, 0)

    def _zs_map(l, i):
        return (jnp.where(l == 3, i, jnp.where(l < 3, 0, last)), 0)

    def _zh_map(l, i):
        return (jnp.where(l == 6, i, 0), 0)

    ws = (W1, W2, W3, W4, W5, W6)
    in_specs = [
        pl.BlockSpec((_BM, x.shape[1]), _x_map),
        pl.BlockSpec((_BM, N), _adj_map),
    ]
    for w in ws:
        in_specs.append(pl.BlockSpec(w.shape, _w_map))

    z_sgae, z_hat, zsb16, mb16, mg16 = pl.pallas_call(
        _mega_kernel,
        grid=(7, _NB),
        in_specs=in_specs,
        out_specs=(pl.BlockSpec((_BM, 128), _zs_map),
                   pl.BlockSpec((_BM, 512), _zh_map),
                   pl.BlockSpec((_BM, 128), _zs_map),
                   pl.BlockSpec((_BM, 128), _zh_map),
                   pl.BlockSpec((_BM, 128), _zh_map)),
        out_shape=(jax.ShapeDtypeStruct((N, 128), _F32),
                   jax.ShapeDtypeStruct((N, 512), _F32),
                   jax.ShapeDtypeStruct((N, 128), _BF16),
                   jax.ShapeDtypeStruct((N, 128), _BF16),
                   jax.ShapeDtypeStruct((N, 128), _BF16)),
        scratch_shapes=[
            pltpu.VMEM((N, N), _BF16),
            pltpu.VMEM((N, 128), _BF16),
            pltpu.VMEM((N, 128), _BF16),
            pltpu.VMEM((N, 128), _BF16),
            pltpu.VMEM((512, 128), _BF16),
            pltpu.VMEM((128, 512), _BF16),
            pltpu.VMEM((128, 128), _BF16),
        ],
    )(x, adj, *ws)

    adj_hat = _recon(zsb16, mg16, mb16)
    return (z_sgae, z_hat, adj_hat)


# speedup vs baseline: 2.1023x; 1.0176x over previous
import jax
import jax.numpy as jnp
from jax.experimental import pallas as pl
from jax.experimental.pallas import tpu as pltpu

N = 4096
_BM = 256
_NB = N // _BM
_RM = 512
_F32 = jnp.float32
_BF16 = jnp.bfloat16


def _mega_kernel(x_ref, adj_ref, w1, w2, w3, w4, w5, w6,
                 zs_out, zh_out, zsb_out, mb_out, mg_out,
                 adjb, ta, tb, zsb, pe, pd, pg):
    l = pl.program_id(0)
    i = pl.program_id(1)
    rows = pl.ds(i * _BM, _BM)

    def dot(a, b):
        return jnp.dot(a, b, preferred_element_type=_F32)

    @pl.when(l == 0)
    def _():
        @pl.when(i == 0)
        def _():
            e = dot(dot(w1[...], w2[...]), w3[...])
            pe[...] = e.astype(_BF16)
            d = dot(dot(w4[...], w5[...]), w6[...])
            pd[...] = d.astype(_BF16)
            g = jax.lax.dot_general(d, d, (((1,), (1,)), ((), ())),
                                    preferred_element_type=_F32)
            pg[...] = (0.5 * g).astype(_BF16)

        xb = x_ref[...].astype(_BF16)
        ta[rows, :] = dot(xb, pe[...]).astype(_BF16)

    @pl.when(l == 1)
    def _():
        ab = adj_ref[...].astype(_BF16)
        adjb[rows, :] = ab
        tb[rows, :] = dot(ab, ta[...]).astype(_BF16)

    @pl.when(l == 2)
    def _():
        ta[rows, :] = dot(adjb[rows, :], tb[...]).astype(_BF16)

    @pl.when(l == 3)
    def _():
        acc = dot(adjb[rows, :], ta[...])
        zs_out[...] = acc
        zb = acc.astype(_BF16)
        zsb_out[...] = zb
        zsb[rows, :] = zb

    @pl.when(l == 4)
    def _():
        tb[rows, :] = dot(adjb[rows, :], zsb[...]).astype(_BF16)

    @pl.when(l == 5)
    def _():
        ta[rows, :] = dot(adjb[rows, :], tb[...]).astype(_BF16)

    @pl.when(l == 6)
    def _():
        m = dot(adjb[rows, :], ta[...])
        mb = m.astype(_BF16)
        zh_out[...] = dot(mb, pd[...])
        mb_out[...] = mb
        mg_out[...] = dot(mb, pg[...]).astype(_BF16)


def _recon_kernel(zs_i, zs_all, mg_i, m_all, o_ref):
    dn = (((1,), (1,)), ((), ()))
    a = jax.lax.dot_general(zs_i[...] * jnp.bfloat16(0.5), zs_all[...], dn,
                            preferred_element_type=_F32)
    b = jax.lax.dot_general(mg_i[...], m_all[...], dn,
                            preferred_element_type=_F32)
    o_ref[...] = 0.5 * (jnp.tanh(a) + jnp.tanh(b)) + 1.0


def _recon(zs, mg, m):
    return pl.pallas_call(
        _recon_kernel,
        grid=(N // _RM,),
        in_specs=[
            pl.BlockSpec((_RM, zs.shape[1]), lambda i: (i, 0)),
            pl.BlockSpec(zs.shape, lambda i: (0, 0)),
            pl.BlockSpec((_RM, mg.shape[1]), lambda i: (i, 0)),
            pl.BlockSpec(m.shape, lambda i: (0, 0)),
        ],
        out_specs=pl.BlockSpec((_RM, N), lambda i: (i, 0)),
        out_shape=jax.ShapeDtypeStruct((N, N), _F32),
    )(zs, zs, mg, m)


def kernel(x, adj, W1, W2, W3, W4, W5, W6):
    last = _NB - 1

    def _x_map(l, i):
        return (jnp.where(l == 0, i, last), 0)

    def _adj_map(l, i):
        return (jnp.where(l == 1, i, jnp.where(l < 1, 0, last)), 0)

    def _w_map(l, i):
        return (0, 0)

    def _zs_map(l, i):
        return (jnp.where(l == 3, i, jnp.where(l < 3, 0, last)), 0)

    def _zh_map(l, i):
        return (jnp.where(l == 6, i, 0), 0)

    ws = (W1, W2, W3, W4, W5, W6)
    in_specs = [
        pl.BlockSpec((_BM, x.shape[1]), _x_map),
        pl.BlockSpec((_BM, N), _adj_map),
    ]
    for w in ws:
        in_specs.append(pl.BlockSpec(w.shape, _w_map))

    z_sgae, z_hat, zsb16, mb16, mg16 = pl.pallas_call(
        _mega_kernel,
        grid=(7, _NB),
        in_specs=in_specs,
        out_specs=(pl.BlockSpec((_BM, 128), _zs_map),
                   pl.BlockSpec((_BM, 512), _zh_map),
                   pl.BlockSpec((_BM, 128), _zs_map),
                   pl.BlockSpec((_BM, 128), _zh_map),
                   pl.BlockSpec((_BM, 128), _zh_map)),
        out_shape=(jax.ShapeDtypeStruct((N, 128), _F32),
                   jax.ShapeDtypeStruct((N, 512), _F32),
                   jax.ShapeDtypeStruct((N, 128), _BF16),
                   jax.ShapeDtypeStruct((N, 128), _BF16),
                   jax.ShapeDtypeStruct((N, 128), _BF16)),
        scratch_shapes=[
            pltpu.VMEM((N, N), _BF16),
            pltpu.VMEM((N, 128), _BF16),
            pltpu.VMEM((N, 128), _BF16),
            pltpu.VMEM((N, 128), _BF16),
            pltpu.VMEM((512, 128), _BF16),
            pltpu.VMEM((128, 512), _BF16),
            pltpu.VMEM((128, 128), _BF16),
        ],
    )(x, adj, *ws)

    adj_hat = _recon(zsb16, mg16, mb16)
    return (z_sgae, z_hat, adj_hat)


# speedup vs baseline: 2.3515x; 1.1186x over previous
import jax
import jax.numpy as jnp
from jax.experimental import pallas as pl
from jax.experimental.pallas import tpu as pltpu

N = 4096
_BM = 512
_NB = N // _BM
_RM = 512
_F32 = jnp.float32
_BF16 = jnp.bfloat16


def _mega_kernel(x_ref, adj_ref, w1, w2, w3, w4, w5, w6,
                 zs_out, zh_out, zsb_out, mb_out, mg_out,
                 adjb, ta, tb, pe, pd, pg):
    l = pl.program_id(0)
    i = pl.program_id(1)
    rows = pl.ds(i * _BM, _BM)

    def dot(a, b):
        return jnp.dot(a, b, preferred_element_type=_F32)

    @pl.when(l == 0)
    def _():
        @pl.when(i == 0)
        def _():
            e = dot(dot(w1[...], w2[...]), w3[...])
            pe[...] = e.astype(_BF16)
            d = dot(dot(w4[...], w5[...]), w6[...])
            pd[...] = d.astype(_BF16)
            g = jax.lax.dot_general(d, d, (((1,), (1,)), ((), ())),
                                    preferred_element_type=_F32)
            pg[...] = (0.5 * g).astype(_BF16)

        ta[rows, :] = dot(x_ref[...], pe[...]).astype(_BF16)

    @pl.when(l == 1)
    def _():
        ab = adj_ref[...].astype(_BF16)
        adjb[rows, :] = ab
        tb[rows, :] = dot(ab, ta[...]).astype(_BF16)

    @pl.when(l == 2)
    def _():
        ta[rows, :] = dot(adjb[rows, :], tb[...]).astype(_BF16)

    @pl.when(l == 3)
    def _():
        acc = dot(adjb[rows, :], ta[...])
        zs_out[...] = acc
        zb = acc.astype(_BF16)
        zsb_out[...] = zb
        tb[rows, :] = zb

    @pl.when(l == 4)
    def _():
        ta[rows, :] = dot(adjb[rows, :], tb[...]).astype(_BF16)

    @pl.when(l == 5)
    def _():
        tb[rows, :] = dot(adjb[rows, :], ta[...]).astype(_BF16)

    @pl.when(l == 6)
    def _():
        m = dot(adjb[rows, :], tb[...])
        mb = m.astype(_BF16)
        zh_out[...] = dot(mb, pd[...])
        mb_out[...] = mb
        mg_out[...] = dot(mb, pg[...]).astype(_BF16)


def _recon_kernel(zs_i, zs_all, mg_i, m_all, o_ref):
    dn = (((1,), (1,)), ((), ()))
    a = jax.lax.dot_general(zs_i[...] * jnp.bfloat16(0.5), zs_all[...], dn,
                            preferred_element_type=_F32)
    b = jax.lax.dot_general(mg_i[...], m_all[...], dn,
                            preferred_element_type=_F32)
    o_ref[...] = 0.5 * (jnp.tanh(a) + jnp.tanh(b)) + 1.0


def _recon(zs, mg, m):
    return pl.pallas_call(
        _recon_kernel,
        grid=(N // _RM,),
        in_specs=[
            pl.BlockSpec((_RM, zs.shape[1]), lambda i: (i, 0)),
            pl.BlockSpec(zs.shape, lambda i: (0, 0)),
            pl.BlockSpec((_RM, mg.shape[1]), lambda i: (i, 0)),
            pl.BlockSpec(m.shape, lambda i: (0, 0)),
        ],
        out_specs=pl.BlockSpec((_RM, N), lambda i: (i, 0)),
        out_shape=jax.ShapeDtypeStruct((N, N), _F32),
    )(zs, zs, mg, m)


def kernel(x, adj, W1, W2, W3, W4, W5, W6):
    last = _NB - 1

    def _x_map(l, i):
        return (jnp.where(l == 0, i, last), 0)

    def _adj_map(l, i):
        return (jnp.where(l == 1, i, jnp.where(l < 1, 0, last)), 0)

    def _w_map(l, i):
        return (0, 0)

    def _zs_map(l, i):
        return (jnp.where(l == 3, i, jnp.where(l < 3, 0, last)), 0)

    def _zh_map(l, i):
        return (jnp.where(l == 6, i, 0), 0)

    ws = (W1, W2, W3, W4, W5, W6)
    in_specs = [
        pl.BlockSpec((_BM, x.shape[1]), _x_map),
        pl.BlockSpec((_BM, N), _adj_map),
    ]
    for w in ws:
        in_specs.append(pl.BlockSpec(w.shape, _w_map))

    z_sgae, z_hat, zsb16, mb16, mg16 = pl.pallas_call(
        _mega_kernel,
        grid=(7, _NB),
        in_specs=in_specs,
        out_specs=(pl.BlockSpec((_BM, 128), _zs_map),
                   pl.BlockSpec((_BM, 512), _zh_map),
                   pl.BlockSpec((_BM, 128), _zs_map),
                   pl.BlockSpec((_BM, 128), _zh_map),
                   pl.BlockSpec((_BM, 128), _zh_map)),
        out_shape=(jax.ShapeDtypeStruct((N, 128), _F32),
                   jax.ShapeDtypeStruct((N, 512), _F32),
                   jax.ShapeDtypeStruct((N, 128), _BF16),
                   jax.ShapeDtypeStruct((N, 128), _BF16),
                   jax.ShapeDtypeStruct((N, 128), _BF16)),
        scratch_shapes=[
            pltpu.VMEM((N, N), _BF16),
            pltpu.VMEM((N, 128), _BF16),
            pltpu.VMEM((N, 128), _BF16),
            pltpu.VMEM((512, 128), _BF16),
            pltpu.VMEM((128, 512), _BF16),
            pltpu.VMEM((128, 128), _BF16),
        ],
    )(x.astype(_BF16), adj, *ws)

    adj_hat = _recon(zsb16, mg16, mb16)
    return (z_sgae, z_hat, adj_hat)
